# baseline (speedup 1.0000x reference)
import jax
import jax.numpy as jnp
from jax import lax
from jax.experimental import pallas as pl
from jax.experimental.pallas import tpu as pltpu
from jax.experimental.pallas import tpu_sc as plsc

V = 1000000
D = 16
H = 64
B = 4096
L = 50

NC, NS = 2, 16
NW = NC * NS
BPW = B // NW
HALF = BPW // 4
CB = 4
RPB = 8 * D


def _gather_body(emb_hbm, row_hbm, sub_hbm, out_hbm,
                 row_v, sub_v, raw_v, out_v, sem, sem2):
    wid = lax.axis_index("s") * NC + lax.axis_index("c")
    base = wid * BPW
    pltpu.sync_copy(row_hbm.at[pl.ds(base, BPW)], row_v)
    pltpu.sync_copy(sub_hbm.at[pl.ds(base, BPW)], sub_v)
    lanes = lax.iota(jnp.int32, 16)

    for half in range(4):
        hb = half * HALF

        def chunk(cb, carry):
            b0 = hb + cb * CB

            def fire(k, c2):
                pltpu.async_copy(emb_hbm.at[row_v.at[b0 + k]],
                                 raw_v.at[k], sem)
                return c2

            lax.fori_loop(0, CB, fire, 0)

            def drain(k, c2):
                pltpu.make_async_copy(emb_hbm.at[row_v.at[b0 + k]],
                                      raw_v.at[k], sem).wait()
                return c2

            lax.fori_loop(0, CB, drain, 0)

            def ext(k, c2):
                kv = jnp.full((16,), k, jnp.int32)
                gv = jnp.full((16,), b0 + k, jnp.int32)
                ov = jnp.full((16,), cb * CB + k, jnp.int32)

                def tblk(tb, c3):
                    t_vec = tb * 16 + lanes
                    msk = t_vec < L
                    s_vec = plsc.load_gather(sub_v, [gv, t_vec], mask=msk)
                    col0 = t_vec * D
                    for d in range(D):
                        vals = plsc.load_gather(
                            raw_v, [kv, t_vec, s_vec + d], mask=msk)
                        plsc.store_scatter(
                            out_v, [ov, col0 + d], vals, mask=msk)
                    return c3

                lax.fori_loop(0, 4, tblk, 0)
                return c2

            lax.fori_loop(0, CB, ext, 0)
            return carry

        lax.fori_loop(0, HALF // CB, chunk, 0)

        pltpu.async_copy(out_v, out_hbm.at[pl.ds(base + hb, HALF)],
                         sem2).wait()


_gather_cache = []


def _gather(emb8, xrow, xsub):
    if not _gather_cache:
        _gather_cache.append(pl.kernel(
            _gather_body,
            out_type=jax.ShapeDtypeStruct((B, L * D), jnp.float32),
            mesh=plsc.VectorSubcoreMesh(
                core_axis_name="c", subcore_axis_name="s",
                num_cores=NC, num_subcores=NS),
            scratch_types=[
                pltpu.VMEM((BPW, L), jnp.int32),
                pltpu.VMEM((BPW, L), jnp.int32),
                pltpu.VMEM((CB, L, RPB), jnp.float32),
                pltpu.VMEM((HALF, L * D), jnp.float32),
                pltpu.SemaphoreType.DMA,
                pltpu.SemaphoreType.DMA,
            ],
            compiler_params=pltpu.CompilerParams(use_tc_tiling_on_sc=False,
                                                 needs_layout_passes=False),
        ))
    return _gather_cache[0](emb8, xrow, xsub)


BB = 2048


def _lstm_mlp_body(e_ref, wih_ref, whh_ref, bg_ref, w1_ref, b1_ref,
                   w2_ref, b2_ref, w3_ref, b3_ref, w4_ref, b4_ref,
                   w5_ref, b5_ref, out_ref, et_ref):
    et_ref[:] = jnp.transpose(e_ref[:], (1, 0))

    def step(t, hc):
        h, c = hc
        xt = et_ref[pl.ds(t * D, D), :]
        g = (jnp.dot(wih_ref[:], xt, preferred_element_type=jnp.float32)
             + jnp.dot(whh_ref[:], h, preferred_element_type=jnp.float32)
             + bg_ref[:])
        i_g = jax.nn.sigmoid(g[0:H])
        f_g = jax.nn.sigmoid(g[H:2 * H])
        g_g = jnp.tanh(g[2 * H:3 * H])
        o_g = jax.nn.sigmoid(g[3 * H:4 * H])
        c = f_g * c + i_g * g_g
        h = o_g * jnp.tanh(c)
        return (h, c)

    h0 = jnp.zeros((H, BB), jnp.float32)
    c0 = jnp.zeros((H, BB), jnp.float32)
    h, _ = lax.fori_loop(0, L, step, (h0, c0))

    a = jax.nn.relu(h)
    a = jax.nn.relu(jnp.dot(w1_ref[:], a, preferred_element_type=jnp.float32)
                    + b1_ref[:])
    a = jax.nn.relu(jnp.dot(w2_ref[:], a, preferred_element_type=jnp.float32)
                    + b2_ref[:])
    a = jax.nn.relu(jnp.dot(w3_ref[:], a, preferred_element_type=jnp.float32)
                    + b3_ref[:])
    a = jax.nn.relu(jnp.dot(w4_ref[:], a, preferred_element_type=jnp.float32)
                    + b4_ref[:])
    a = jax.nn.sigmoid(jnp.dot(w5_ref[:], a, preferred_element_type=jnp.float32)
                       + b5_ref[:])
    out_ref[:] = a


def _full(shape):
    return pl.BlockSpec(shape, lambda *_: tuple(0 for _ in shape))


def _lstm_mlp(e, wih, whh, bg, w1, b1, w2, b2, w3, b3, w4, b4, w5, b5,
              interpret=False):
    return pl.pallas_call(
        _lstm_mlp_body,
        grid=(B // BB,),
        scratch_shapes=[pltpu.VMEM((L * D, BB), jnp.float32)],
        in_specs=[
            pl.BlockSpec((BB, L * D), lambda i: (i, 0)),
            _full(wih.shape), _full(whh.shape), _full(bg.shape),
            _full(w1.shape), _full(b1.shape),
            _full(w2.shape), _full(b2.shape),
            _full(w3.shape), _full(b3.shape),
            _full(w4.shape), _full(b4.shape),
            _full(w5.shape), _full(b5.shape),
        ],
        out_specs=pl.BlockSpec((1, BB), lambda i: (0, i)),
        out_shape=jax.ShapeDtypeStruct((1, B), jnp.float32),
        interpret=interpret,
    )(e, wih, whh, bg, w1, b1, w2, b2, w3, b3, w4, b4, w5, b5)


def kernel(x, emb, W_ih, W_hh, b_ih, b_hh, W1, b1, W2, b2, W3, b3, W4, b4,
           W5, b5):
    xi = x.astype(jnp.int32)
    emb8 = lax.optimization_barrier(emb.reshape(V // 8, RPB))
    e = _gather(emb8, xi >> 3, (xi & 7) * D)
    bg = (b_ih + b_hh).reshape(4 * H, 1)
    out = _lstm_mlp(
        e, W_ih, W_hh, bg,
        W1, b1.reshape(-1, 1), W2, b2.reshape(-1, 1),
        W3, b3.reshape(-1, 1), W4, b4.reshape(-1, 1),
        W5, b5.reshape(1, 1))
    return out.reshape(B, 1)


# speedup vs baseline: 1.1890x; 1.1890x over previous
import jax
import jax.numpy as jnp
from jax import lax
from jax.experimental import pallas as pl
from jax.experimental.pallas import tpu as pltpu
from jax.experimental.pallas import tpu_sc as plsc

V = 1000000
D = 16
H = 64
B = 4096
L = 50

NC, NS = 2, 16
NW = NC * NS
ROWS = B * L
RPW = ROWS // NW
CHUNK = 128
NCHUNK = RPW // CHUNK


def _gather_body(emb_hbm, idx_hbm, out_hbm, idx_v, rows_v, sem):
    wid = lax.axis_index("s") * NC + lax.axis_index("c")
    pltpu.sync_copy(idx_hbm.at[wid], idx_v)

    def fire(j, carry):
        pltpu.async_copy(emb_hbm.at[idx_v.at[j]], rows_v.at[j], sem)
        return carry

    lax.fori_loop(0, NCHUNK, fire, 0)

    def drain(j, carry):
        pltpu.make_async_copy(emb_hbm.at[idx_v.at[j]], rows_v.at[j],
                              sem).wait()
        return carry

    lax.fori_loop(0, NCHUNK, drain, 0)
    pltpu.sync_copy(rows_v, out_hbm.at[pl.ds(wid * NCHUNK, NCHUNK)])


_gather_cache = []


def _gather(emb, idx):
    if not _gather_cache:
        _gather_cache.append(pl.kernel(
            _gather_body,
            out_type=jax.ShapeDtypeStruct((NW * NCHUNK, CHUNK, D),
                                          emb.dtype),
            mesh=plsc.VectorSubcoreMesh(
                core_axis_name="c", subcore_axis_name="s",
                num_cores=NC, num_subcores=NS),
            scratch_types=[
                pltpu.VMEM((NCHUNK, CHUNK), jnp.int32),
                pltpu.VMEM((NCHUNK, CHUNK, D), emb.dtype),
                pltpu.SemaphoreType.DMA,
            ],
            compiler_params=pltpu.CompilerParams(use_tc_tiling_on_sc=False),
        ))
    return _gather_cache[0](emb, idx)


BB = 2048


def _lstm_mlp_body(e_ref, wih_ref, whh_ref, bg_ref, w1_ref, b1_ref,
                   w2_ref, b2_ref, w3_ref, b3_ref, w4_ref, b4_ref,
                   w5_ref, b5_ref, out_ref, et_ref):
    et_ref[:] = jnp.transpose(e_ref[:], (1, 0))

    def step(t, hc):
        h, c = hc
        xt = et_ref[pl.ds(t * D, D), :]
        g = (jnp.dot(wih_ref[:], xt, preferred_element_type=jnp.float32)
             + jnp.dot(whh_ref[:], h, preferred_element_type=jnp.float32)
             + bg_ref[:])
        i_g = jax.nn.sigmoid(g[0:H])
        f_g = jax.nn.sigmoid(g[H:2 * H])
        g_g = jnp.tanh(g[2 * H:3 * H])
        o_g = jax.nn.sigmoid(g[3 * H:4 * H])
        c = f_g * c + i_g * g_g
        h = o_g * jnp.tanh(c)
        return (h, c)

    h0 = jnp.zeros((H, BB), jnp.float32)
    c0 = jnp.zeros((H, BB), jnp.float32)
    h, _ = lax.fori_loop(0, L, step, (h0, c0))

    a = jax.nn.relu(h)
    a = jax.nn.relu(jnp.dot(w1_ref[:], a, preferred_element_type=jnp.float32)
                    + b1_ref[:])
    a = jax.nn.relu(jnp.dot(w2_ref[:], a, preferred_element_type=jnp.float32)
                    + b2_ref[:])
    a = jax.nn.relu(jnp.dot(w3_ref[:], a, preferred_element_type=jnp.float32)
                    + b3_ref[:])
    a = jax.nn.relu(jnp.dot(w4_ref[:], a, preferred_element_type=jnp.float32)
                    + b4_ref[:])
    a = jax.nn.sigmoid(jnp.dot(w5_ref[:], a, preferred_element_type=jnp.float32)
                       + b5_ref[:])
    out_ref[:] = a


def _full(shape):
    return pl.BlockSpec(shape, lambda *_: tuple(0 for _ in shape))


def _lstm_mlp(e, wih, whh, bg, w1, b1, w2, b2, w3, b3, w4, b4, w5, b5,
              interpret=False):
    return pl.pallas_call(
        _lstm_mlp_body,
        grid=(B // BB,),
        scratch_shapes=[pltpu.VMEM((L * D, BB), jnp.float32)],
        in_specs=[
            pl.BlockSpec((BB, L * D), lambda i: (i, 0)),
            _full(wih.shape), _full(whh.shape), _full(bg.shape),
            _full(w1.shape), _full(b1.shape),
            _full(w2.shape), _full(b2.shape),
            _full(w3.shape), _full(b3.shape),
            _full(w4.shape), _full(b4.shape),
            _full(w5.shape), _full(b5.shape),
        ],
        out_specs=pl.BlockSpec((1, BB), lambda i: (0, i)),
        out_shape=jax.ShapeDtypeStruct((1, B), jnp.float32),
        interpret=interpret,
    )(e, wih, whh, bg, w1, b1, w2, b2, w3, b3, w4, b4, w5, b5)


def kernel(x, emb, W_ih, W_hh, b_ih, b_hh, W1, b1, W2, b2, W3, b3, W4, b4,
           W5, b5):
    idx = x.astype(jnp.int32).reshape(NW, NCHUNK, CHUNK)
    e = _gather(emb, idx)
    e = e.reshape(B, L * D)
    bg = (b_ih + b_hh).reshape(4 * H, 1)
    out = _lstm_mlp(
        e, W_ih, W_hh, bg,
        W1, b1.reshape(-1, 1), W2, b2.reshape(-1, 1),
        W3, b3.reshape(-1, 1), W4, b4.reshape(-1, 1),
        W5, b5.reshape(1, 1))
    return out.reshape(B, 1)
